# baseline (speedup 1.0000x reference)
import functools

import jax
import jax.numpy as jnp
from jax import lax
from jax.experimental import pallas as pl
from jax.experimental.pallas import tpu as pltpu
from jax.experimental.pallas import tpu_sc as plsc

N_CORE = 20000
N_AUX = 5000
AUX_OFF = 20096
GAP = AUX_OFF - N_CORE
BINS_PAD = 26624
D = 256
N_IDS = 1024000
NUM_WORKERS = 32
LANES = 16
ID_R = 1000

_mesh = plsc.VectorSubcoreMesh(core_axis_name="c", subcore_axis_name="s")

NUM_SUB = 16
SL = BINS_PAD // NUM_SUB


@functools.partial(
    pl.kernel,
    out_type=jax.ShapeDtypeStruct((2, BINS_PAD), jnp.float32),
    mesh=_mesh,
    scratch_types=[
        pltpu.VMEM((8, 32, 32), jnp.int32),
        pltpu.VMEM((BINS_PAD,), jnp.float32),
        pltpu.VMEM((NUM_SUB, SL), jnp.float32),
        pltpu.VMEM_SHARED((NUM_SUB, BINS_PAD), jnp.float32),
        pltpu.SemaphoreType.DMA,
    ],
    compiler_params=pltpu.CompilerParams(needs_layout_passes=False),
)
def _histogram(ids_hbm, out_hbm, ids_v, counts_v, red_v, shared, sem):
    cid = lax.axis_index("c")
    sid = lax.axis_index("s")
    wid = sid * 2 + cid
    r0 = (wid * ID_R) // NUM_WORKERS
    nr = ((wid + 1) * ID_R) // NUM_WORKERS - r0

    zeros = jnp.zeros((LANES,), jnp.float32)

    @plsc.parallel_loop(0, BINS_PAD // LANES, unroll=8)
    def _zero(i):
        counts_v[pl.ds(i * LANES, LANES)] = zeros

    ones = jnp.ones((LANES,), jnp.float32)

    def _scatter_rows(lo):
        def _row(r, carry):
            for c in range(32):
                for h in range(2):
                    idx = ids_v[r, c, pl.ds(h * LANES, LANES)]
                    idx = jnp.where(idx >= N_CORE, idx + GAP, idx)
                    plsc.addupdate_scatter(counts_v, [idx], ones)
            return carry
        lax.fori_loop(lo, 8, _row, 0)

    for k in range(3):
        pltpu.async_copy(ids_hbm.at[pl.ds(r0 + k * 8, 8)], ids_v, sem).wait()
        _scatter_rows(0)
    pltpu.async_copy(ids_hbm.at[pl.ds(r0 + nr - 8, 8)], ids_v, sem).wait()
    _scatter_rows(32 - nr)

    pltpu.sync_copy(counts_v, shared.at[sid])
    plsc.subcore_barrier()
    base = sid * SL
    pltpu.sync_copy(shared.at[:, pl.ds(base, SL)], red_v)

    @plsc.parallel_loop(0, SL // LANES, unroll=2)
    def _red(i):
        acc = red_v[0, pl.ds(i * LANES, LANES)]
        for r in range(1, NUM_SUB):
            acc = acc + red_v[r, pl.ds(i * LANES, LANES)]
        counts_v[pl.ds(i * LANES, LANES)] = acc

    pltpu.sync_copy(counts_v.at[pl.ds(0, SL)],
                    out_hbm.at[cid, pl.ds(base, SL)])


_CB = 4096
_AB = 1024
_STEPS = 5


def _matvec_body(cnt_ref, core_ref, aux_ref, out_ref, acc_ref):
    j = pl.program_id(0)

    @pl.when(j == 0)
    def _init():
        acc_ref[...] = jnp.zeros_like(acc_ref)

    cc = cnt_ref[:, pl.ds(j * _CB, _CB)]
    ca = cnt_ref[:, pl.ds(AUX_OFF + j * _AB, _AB)]

    core_rows = j * _CB + lax.broadcasted_iota(jnp.int32, (_CB, 1), 0)
    core = jnp.where(core_rows < N_CORE, core_ref[...], 0.0)
    aux_rows = j * _AB + lax.broadcasted_iota(jnp.int32, (_AB, 1), 0)
    aux = jnp.where(aux_rows < N_AUX, aux_ref[...], 0.0)

    acc = lax.dot_general(
        cc, core, (((1,), (0,)), ((), ())),
        precision=lax.Precision.HIGHEST,
        preferred_element_type=jnp.float32,
    )
    acc = acc + lax.dot_general(
        ca, aux, (((1,), (0,)), ((), ())),
        precision=lax.Precision.HIGHEST,
        preferred_element_type=jnp.float32,
    )
    acc_ref[...] += acc

    @pl.when(j == _STEPS - 1)
    def _fin():
        out_ref[...] = (acc_ref[0:1, :] + acc_ref[1:2, :]) * (1.0 / N_IDS)


def _weighted_sum(counts, core_feats, aux_feats):
    return pl.pallas_call(
        _matvec_body,
        grid=(_STEPS,),
        in_specs=[
            pl.BlockSpec((2, BINS_PAD), lambda j: (0, 0)),
            pl.BlockSpec((_CB, D), lambda j: (j, 0)),
            pl.BlockSpec((_AB, D), lambda j: (j, 0)),
        ],
        out_specs=pl.BlockSpec((1, D), lambda j: (0, 0)),
        out_shape=jax.ShapeDtypeStruct((1, D), jnp.float32),
        scratch_shapes=[pltpu.VMEM((2, D), jnp.float32)],
    )(counts, core_feats, aux_feats)


def kernel(core_feats, aux_feats, id_map):
    counts = _histogram(id_map)
    return _weighted_sum(counts, core_feats, aux_feats)


# speedup vs baseline: 1.2897x; 1.2897x over previous
import functools

import jax
import jax.numpy as jnp
from jax import lax
from jax.experimental import pallas as pl
from jax.experimental.pallas import tpu as pltpu
from jax.experimental.pallas import tpu_sc as plsc

N_CORE = 20000
N_AUX = 5000
AUX_OFF = 20096
GAP = AUX_OFF - N_CORE
BINS_PAD = 26624
D = 256
N_IDS = 1024000
NUM_WORKERS = 32
LANES = 16
ID_R = 1000

_mesh = plsc.VectorSubcoreMesh(core_axis_name="c", subcore_axis_name="s")

NUM_SUB = 16
SL = BINS_PAD // NUM_SUB


@functools.partial(
    pl.kernel,
    out_type=jax.ShapeDtypeStruct((2, BINS_PAD), jnp.float32),
    mesh=_mesh,
    scratch_types=[
        pltpu.VMEM((2, 8, 1024), jnp.int32),
        pltpu.VMEM((BINS_PAD,), jnp.float32),
        pltpu.VMEM((NUM_SUB, SL), jnp.float32),
        pltpu.VMEM_SHARED((NUM_SUB, BINS_PAD), jnp.float32),
        pltpu.SemaphoreType.DMA,
        pltpu.SemaphoreType.DMA,
    ],
    compiler_params=pltpu.CompilerParams(needs_layout_passes=False),
)
def _histogram(ids_hbm, out_hbm, ids_v, counts_v, red_v, shared, sem0, sem1):
    cid = lax.axis_index("c")
    sid = lax.axis_index("s")
    wid = sid * 2 + cid
    c0 = (wid * (ID_R // 8)) // NUM_WORKERS
    c1 = ((wid + 1) * (ID_R // 8)) // NUM_WORKERS
    sems = (sem0, sem1)

    def _start(k, buf):
        return pltpu.async_copy(
            ids_hbm.at[pl.ds(k * 8, 8)], ids_v.at[buf], sems[buf]
        )

    cp0 = _start(c0, 0)

    zeros = jnp.zeros((LANES,), jnp.float32)

    @plsc.parallel_loop(0, BINS_PAD // LANES, unroll=8)
    def _zero(i):
        counts_v[pl.ds(i * LANES, LANES)] = zeros

    ones = jnp.ones((LANES,), jnp.float32)

    def _scatter_chunk(buf):
        def _row(r, carry):
            for h in range(1024 // LANES):
                idx = ids_v[buf, r, pl.ds(h * LANES, LANES)]
                idx = jnp.where(idx >= N_CORE, idx + GAP, idx)
                plsc.addupdate_scatter(counts_v, [idx], ones)
            return carry
        lax.fori_loop(0, 8, _row, 0)

    cp1 = _start(c0 + 1, 1)
    cp0.wait()
    _scatter_chunk(0)
    cp0 = _start(c0 + 2, 0)
    cp1.wait()
    _scatter_chunk(1)
    cp1 = _start(c1 - 1, 1)
    cp0.wait()
    _scatter_chunk(0)
    cp1.wait()

    @pl.when(c1 - c0 == 4)
    def _tail():
        _scatter_chunk(1)

    pltpu.sync_copy(counts_v, shared.at[sid])
    plsc.subcore_barrier()
    base = sid * SL
    pltpu.sync_copy(shared.at[:, pl.ds(base, SL)], red_v)

    @plsc.parallel_loop(0, SL // LANES, unroll=2)
    def _red(i):
        acc = red_v[0, pl.ds(i * LANES, LANES)]
        for r in range(1, NUM_SUB):
            acc = acc + red_v[r, pl.ds(i * LANES, LANES)]
        counts_v[pl.ds(i * LANES, LANES)] = acc

    pltpu.sync_copy(counts_v.at[pl.ds(0, SL)],
                    out_hbm.at[cid, pl.ds(base, SL)])


_CB = 4096
_AB = 1024
_STEPS = 5


def _matvec_body(cnt_ref, core_ref, aux_ref, out_ref, acc_ref):
    j = pl.program_id(0)

    @pl.when(j == 0)
    def _init():
        acc_ref[...] = jnp.zeros_like(acc_ref)

    cc = cnt_ref[:, pl.ds(j * _CB, _CB)]
    ca = cnt_ref[:, pl.ds(AUX_OFF + j * _AB, _AB)]

    core_rows = j * _CB + lax.broadcasted_iota(jnp.int32, (_CB, 1), 0)
    core = jnp.where(core_rows < N_CORE, core_ref[...], 0.0)
    aux_rows = j * _AB + lax.broadcasted_iota(jnp.int32, (_AB, 1), 0)
    aux = jnp.where(aux_rows < N_AUX, aux_ref[...], 0.0)

    acc = lax.dot_general(
        cc, core, (((1,), (0,)), ((), ())),
        precision=lax.Precision.HIGHEST,
        preferred_element_type=jnp.float32,
    )
    acc = acc + lax.dot_general(
        ca, aux, (((1,), (0,)), ((), ())),
        precision=lax.Precision.HIGHEST,
        preferred_element_type=jnp.float32,
    )
    acc_ref[...] += acc

    @pl.when(j == _STEPS - 1)
    def _fin():
        out_ref[...] = (acc_ref[0:1, :] + acc_ref[1:2, :]) * (1.0 / N_IDS)


def _weighted_sum(counts, core_feats, aux_feats):
    return pl.pallas_call(
        _matvec_body,
        grid=(_STEPS,),
        in_specs=[
            pl.BlockSpec((2, BINS_PAD), lambda j: (0, 0)),
            pl.BlockSpec((_CB, D), lambda j: (j, 0)),
            pl.BlockSpec((_AB, D), lambda j: (j, 0)),
        ],
        out_specs=pl.BlockSpec((1, D), lambda j: (0, 0)),
        out_shape=jax.ShapeDtypeStruct((1, D), jnp.float32),
        scratch_shapes=[pltpu.VMEM((2, D), jnp.float32)],
    )(counts, core_feats, aux_feats)


def kernel(core_feats, aux_feats, id_map):
    ids = id_map.reshape(ID_R, 1024)
    counts = _histogram(ids)
    return _weighted_sum(counts, core_feats, aux_feats)


# speedup vs baseline: 1.3731x; 1.0647x over previous
import functools

import jax
import jax.numpy as jnp
from jax import lax
from jax.experimental import pallas as pl
from jax.experimental.pallas import tpu as pltpu
from jax.experimental.pallas import tpu_sc as plsc

N_CORE = 20000
N_AUX = 5000
AUX_OFF = 20096
GAP = AUX_OFF - N_CORE
BINS_PAD = 26624
D = 256
N_IDS = 1024000
NUM_WORKERS = 32
LANES = 16
ID_R = 1000

_mesh = plsc.VectorSubcoreMesh(core_axis_name="c", subcore_axis_name="s")

NUM_SUB = 16
SL = BINS_PAD // NUM_SUB


@functools.partial(
    pl.kernel,
    out_type=jax.ShapeDtypeStruct((2, BINS_PAD), jnp.float32),
    mesh=_mesh,
    scratch_types=[
        pltpu.VMEM((2, 8, 1024), jnp.int32),
        pltpu.VMEM((BINS_PAD,), jnp.float32),
        pltpu.VMEM((NUM_SUB, SL), jnp.float32),
        pltpu.VMEM_SHARED((NUM_SUB, BINS_PAD), jnp.float32),
        pltpu.SemaphoreType.DMA,
        pltpu.SemaphoreType.DMA,
    ],
    compiler_params=pltpu.CompilerParams(needs_layout_passes=False),
)
def _histogram(ids_hbm, out_hbm, ids_v, counts_v, red_v, shared, sem0, sem1):
    cid = lax.axis_index("c")
    sid = lax.axis_index("s")
    wid = sid * 2 + cid
    c0 = (wid * (ID_R // 8)) // NUM_WORKERS
    c1 = ((wid + 1) * (ID_R // 8)) // NUM_WORKERS
    sems = (sem0, sem1)

    def _start(k, buf):
        return pltpu.async_copy(
            ids_hbm.at[pl.ds(k * 8, 8)], ids_v.at[buf], sems[buf]
        )

    cp0 = _start(c0, 0)

    zeros = jnp.zeros((LANES,), jnp.float32)

    @plsc.parallel_loop(0, BINS_PAD // LANES, unroll=8)
    def _zero(i):
        counts_v[pl.ds(i * LANES, LANES)] = zeros

    ones = jnp.ones((LANES,), jnp.float32)

    def _scatter_chunk(buf):
        @plsc.parallel_loop(0, 8, unroll=1)
        def _row(r):
            for h in range(1024 // LANES):
                idx = ids_v[buf, r, pl.ds(h * LANES, LANES)]
                idx = jnp.where(idx >= N_CORE, idx + GAP, idx)
                plsc.addupdate_scatter(counts_v, [idx], ones)

    cp1 = _start(c0 + 1, 1)
    cp0.wait()
    _scatter_chunk(0)
    cp0 = _start(c0 + 2, 0)
    cp1.wait()
    _scatter_chunk(1)
    cp1 = _start(c1 - 1, 1)
    cp0.wait()
    _scatter_chunk(0)
    cp1.wait()

    @pl.when(c1 - c0 == 4)
    def _tail():
        _scatter_chunk(1)

    pltpu.sync_copy(counts_v, shared.at[sid])
    plsc.subcore_barrier()
    base = sid * SL
    pltpu.sync_copy(shared.at[:, pl.ds(base, SL)], red_v)

    @plsc.parallel_loop(0, SL // LANES, unroll=2)
    def _red(i):
        acc = red_v[0, pl.ds(i * LANES, LANES)]
        for r in range(1, NUM_SUB):
            acc = acc + red_v[r, pl.ds(i * LANES, LANES)]
        counts_v[pl.ds(i * LANES, LANES)] = acc

    pltpu.sync_copy(counts_v.at[pl.ds(0, SL)],
                    out_hbm.at[cid, pl.ds(base, SL)])


_CB = 4096
_AB = 1024
_STEPS = 5


def _matvec_body(cnt_ref, core_ref, aux_ref, out_ref, acc_ref):
    j = pl.program_id(0)

    @pl.when(j == 0)
    def _init():
        acc_ref[...] = jnp.zeros_like(acc_ref)

    cc = cnt_ref[:, pl.ds(j * _CB, _CB)]
    ca = cnt_ref[:, pl.ds(AUX_OFF + j * _AB, _AB)]

    core_rows = j * _CB + lax.broadcasted_iota(jnp.int32, (_CB, 1), 0)
    core = jnp.where(core_rows < N_CORE, core_ref[...], 0.0)
    aux_rows = j * _AB + lax.broadcasted_iota(jnp.int32, (_AB, 1), 0)
    aux = jnp.where(aux_rows < N_AUX, aux_ref[...], 0.0)

    acc = lax.dot_general(
        cc, core, (((1,), (0,)), ((), ())),
        precision=lax.Precision.HIGHEST,
        preferred_element_type=jnp.float32,
    )
    acc = acc + lax.dot_general(
        ca, aux, (((1,), (0,)), ((), ())),
        precision=lax.Precision.HIGHEST,
        preferred_element_type=jnp.float32,
    )
    acc_ref[...] += acc

    @pl.when(j == _STEPS - 1)
    def _fin():
        out_ref[...] = (acc_ref[0:1, :] + acc_ref[1:2, :]) * (1.0 / N_IDS)


def _weighted_sum(counts, core_feats, aux_feats):
    return pl.pallas_call(
        _matvec_body,
        grid=(_STEPS,),
        in_specs=[
            pl.BlockSpec((2, BINS_PAD), lambda j: (0, 0)),
            pl.BlockSpec((_CB, D), lambda j: (j, 0)),
            pl.BlockSpec((_AB, D), lambda j: (j, 0)),
        ],
        out_specs=pl.BlockSpec((1, D), lambda j: (0, 0)),
        out_shape=jax.ShapeDtypeStruct((1, D), jnp.float32),
        scratch_shapes=[pltpu.VMEM((2, D), jnp.float32)],
    )(counts, core_feats, aux_feats)


def kernel(core_feats, aux_feats, id_map):
    ids = id_map.reshape(ID_R, 1024)
    counts = _histogram(ids)
    return _weighted_sum(counts, core_feats, aux_feats)


# speedup vs baseline: 1.6089x; 1.1717x over previous
import functools

import jax
import jax.numpy as jnp
from jax import lax
from jax.experimental import pallas as pl
from jax.experimental.pallas import tpu as pltpu
from jax.experimental.pallas import tpu_sc as plsc

N_CORE = 20000
N_AUX = 5000
AUX_OFF = 20096
GAP = AUX_OFF - N_CORE
BINS_PAD = 26624
D = 256
N_IDS = 1024000
NUM_WORKERS = 32
LANES = 16
ID_R = 1000

_mesh = plsc.VectorSubcoreMesh(core_axis_name="c", subcore_axis_name="s")

NUM_SUB = 16
SL = BINS_PAD // NUM_SUB


@functools.partial(
    pl.kernel,
    out_type=jax.ShapeDtypeStruct((2, BINS_PAD), jnp.float32),
    mesh=_mesh,
    scratch_types=[
        pltpu.VMEM((32, 1024), jnp.int32),
        pltpu.VMEM((BINS_PAD,), jnp.float32),
        pltpu.VMEM((NUM_SUB, SL), jnp.float32),
        pltpu.VMEM_SHARED((NUM_SUB, BINS_PAD), jnp.float32),
        pltpu.SemaphoreType.DMA,
    ],
    compiler_params=pltpu.CompilerParams(needs_layout_passes=False),
)
def _histogram(ids_hbm, out_hbm, ids_v, counts_v, red_v, shared, sem):
    cid = lax.axis_index("c")
    sid = lax.axis_index("s")
    wid = sid * 2 + cid
    c0 = (wid * (ID_R // 8)) // NUM_WORKERS
    c1 = ((wid + 1) * (ID_R // 8)) // NUM_WORKERS
    nch = c1 - c0

    cps = [pltpu.async_copy(ids_hbm.at[pl.ds((c0 + k) * 8, 8)],
                            ids_v.at[pl.ds(k * 8, 8)], sem)
           for k in range(3)]

    @pl.when(nch == 4)
    def _fetch4():
        pltpu.async_copy(ids_hbm.at[pl.ds((c0 + 3) * 8, 8)],
                         ids_v.at[pl.ds(24, 8)], sem)

    zeros = jnp.zeros((LANES,), jnp.float32)

    @plsc.parallel_loop(0, BINS_PAD // LANES, unroll=8)
    def _zero(i):
        counts_v[pl.ds(i * LANES, LANES)] = zeros

    for cp in cps:
        cp.wait()

    @pl.when(nch == 4)
    def _wait4():
        pltpu.make_async_copy(ids_hbm.at[pl.ds((c0 + 3) * 8, 8)],
                              ids_v.at[pl.ds(24, 8)], sem).wait()

    ones = jnp.ones((LANES,), jnp.float32)
    nv = nch * (8 * 1024 // LANES)

    @plsc.parallel_loop(0, nv, unroll=8)
    def _scat(i):
        r = i >> 6
        h = i & 63
        idx = ids_v[r, pl.ds(h * LANES, LANES)]
        idx = jnp.where(idx >= N_CORE, idx + GAP, idx)
        plsc.addupdate_scatter(counts_v, [idx], ones)

    pltpu.sync_copy(counts_v, shared.at[sid])
    plsc.subcore_barrier()
    base = sid * SL
    pltpu.sync_copy(shared.at[:, pl.ds(base, SL)], red_v)

    @plsc.parallel_loop(0, SL // LANES, unroll=2)
    def _red(i):
        acc = red_v[0, pl.ds(i * LANES, LANES)]
        for r in range(1, NUM_SUB):
            acc = acc + red_v[r, pl.ds(i * LANES, LANES)]
        counts_v[pl.ds(i * LANES, LANES)] = acc

    pltpu.sync_copy(counts_v.at[pl.ds(0, SL)],
                    out_hbm.at[cid, pl.ds(base, SL)])


_CB = 4096
_AB = 1024
_STEPS = 5


def _matvec_body(cnt_ref, core_ref, aux_ref, out_ref, acc_ref):
    j = pl.program_id(0)

    @pl.when(j == 0)
    def _init():
        acc_ref[...] = jnp.zeros_like(acc_ref)

    cc = cnt_ref[:, pl.ds(j * _CB, _CB)]
    ca = cnt_ref[:, pl.ds(AUX_OFF + j * _AB, _AB)]

    core_rows = j * _CB + lax.broadcasted_iota(jnp.int32, (_CB, 1), 0)
    core = jnp.where(core_rows < N_CORE, core_ref[...], 0.0)
    aux_rows = j * _AB + lax.broadcasted_iota(jnp.int32, (_AB, 1), 0)
    aux = jnp.where(aux_rows < N_AUX, aux_ref[...], 0.0)

    acc = lax.dot_general(
        cc, core, (((1,), (0,)), ((), ())),
        precision=lax.Precision.HIGHEST,
        preferred_element_type=jnp.float32,
    )
    acc = acc + lax.dot_general(
        ca, aux, (((1,), (0,)), ((), ())),
        precision=lax.Precision.HIGHEST,
        preferred_element_type=jnp.float32,
    )
    acc_ref[...] += acc

    @pl.when(j == _STEPS - 1)
    def _fin():
        out_ref[...] = (acc_ref[0:1, :] + acc_ref[1:2, :]) * (1.0 / N_IDS)


def _weighted_sum(counts, core_feats, aux_feats):
    return pl.pallas_call(
        _matvec_body,
        grid=(_STEPS,),
        in_specs=[
            pl.BlockSpec((2, BINS_PAD), lambda j: (0, 0)),
            pl.BlockSpec((_CB, D), lambda j: (j, 0)),
            pl.BlockSpec((_AB, D), lambda j: (j, 0)),
        ],
        out_specs=pl.BlockSpec((1, D), lambda j: (0, 0)),
        out_shape=jax.ShapeDtypeStruct((1, D), jnp.float32),
        scratch_shapes=[pltpu.VMEM((2, D), jnp.float32)],
    )(counts, core_feats, aux_feats)


def kernel(core_feats, aux_feats, id_map):
    ids = id_map.reshape(ID_R, 1024)
    counts = _histogram(ids)
    return _weighted_sum(counts, core_feats, aux_feats)
